# Initial kernel scaffold; baseline (speedup 1.0000x reference)
import jax
import jax.numpy as jnp
from jax.experimental import pallas as pl


def kernel(queries, candidates):
    raise NotImplementedError("write your pallas kernel here")


# speedup vs baseline: 4.1428x; 4.1428x over previous
import functools

import jax
import jax.numpy as jnp
from jax import lax
from jax.experimental import pallas as pl
from jax.experimental.pallas import tpu as pltpu
from jax.experimental.pallas import tpu_sc as plsc

Q = 4096
D = 128
N = 100000
K_OUT = 10

CHUNK = 128
NPAD = 102400
M = NPAD // CHUNK
K_CHUNKS = 16

BQ1 = 1024
BN1 = 2048
NB1 = NPAD // BN1
CPB = BN1 // CHUNK

BQ2 = 512
BQ4 = 256

NEG = -1.0e30
BIG = 2**30


def _k1_body(q_ref, c_ref, s_ref, cm_ref):
    ni = pl.program_id(1)
    s = lax.dot_general(
        q_ref[...], c_ref[...],
        dimension_numbers=(((1,), (1,)), ((), ())),
        preferred_element_type=jnp.float32,
    )
    col = ni * BN1 + lax.broadcasted_iota(jnp.int32, (BQ1, BN1), 1)
    s = jnp.where(col < N, s, NEG)
    s_ref[...] = s
    cm_ref[0] = jnp.max(s.reshape(BQ1, CPB, CHUNK), axis=-1)


def _k1(queries, cand_pad):
    return pl.pallas_call(
        _k1_body,
        grid=(Q // BQ1, NB1),
        in_specs=[
            pl.BlockSpec((BQ1, D), lambda qi, ni: (qi, 0)),
            pl.BlockSpec((BN1, D), lambda qi, ni: (ni, 0)),
        ],
        out_specs=[
            pl.BlockSpec((BQ1, BN1), lambda qi, ni: (qi, ni)),
            pl.BlockSpec((1, BQ1, CPB), lambda qi, ni: (ni, qi, 0)),
        ],
        out_shape=[
            jax.ShapeDtypeStruct((Q, NPAD), jnp.float32),
            jax.ShapeDtypeStruct((NB1, Q, CPB), jnp.float32),
        ],
    )(queries, cand_pad)


def _k2_body(cm_ref, cid_ref):
    cm = cm_ref[...]
    gid0 = lax.broadcasted_iota(jnp.int32, (M, BQ2), 0)
    lane16 = lax.broadcasted_iota(jnp.int32, (BQ2, K_CHUNKS), 1)
    cid_acc = jnp.zeros((BQ2, K_CHUNKS), jnp.int32)
    for p in range(K_CHUNKS):
        m = jnp.max(cm, axis=0)
        sel = jnp.where(cm == m[None, :], gid0, BIG)
        am = jnp.min(sel, axis=0)
        cid_acc = jnp.where(lane16 == p, am[:, None], cid_acc)
        cm = jnp.where(gid0 == am[None, :], NEG, cm)
    cid_ref[...] = cid_acc


def _k2(cm2):
    return pl.pallas_call(
        _k2_body,
        grid=(Q // BQ2,),
        in_specs=[pl.BlockSpec((M, BQ2), lambda qi: (0, qi))],
        out_specs=pl.BlockSpec((BQ2, K_CHUNKS), lambda qi: (qi, 0)),
        out_shape=jax.ShapeDtypeStruct((Q, K_CHUNKS), jnp.int32),
    )(cm2)


NW = 32
QPW = Q // NW
RPW = QPW * K_CHUNKS
NG = 8
GR = RPW // NG
MPG = QPW // NG


CIDROWS = QPW * K_CHUNKS // 128


def _k3_body(s_hbm, cid_hbm, out_hbm, cid_v, idx_v, rows_v, sem):
    wid = lax.axis_index("s") * 2 + lax.axis_index("c")
    qlo = wid * QPW
    pltpu.sync_copy(cid_hbm.at[pl.ds(wid * CIDROWS, CIDROWS)], cid_v)

    def fill(i, carry):
        flat = cid_v[i // 8, pl.ds((i % 8) * K_CHUNKS, K_CHUNKS)]
        flat = flat + (qlo + i) * M
        g = i // MPG
        off = (i % MPG) * K_CHUNKS
        idx_v[g, pl.ds(off, K_CHUNKS)] = flat
        return carry

    lax.fori_loop(0, QPW, fill, 0)
    for g in range(NG):
        pltpu.async_copy(s_hbm.at[idx_v.at[g]], rows_v, sem).wait()
        pltpu.sync_copy(rows_v,
                        out_hbm.at[pl.ds(wid * RPW + g * GR, GR)])


def _k3(s_flat, cid):
    mesh = plsc.VectorSubcoreMesh(core_axis_name="c", subcore_axis_name="s")
    kern = functools.partial(
        pl.kernel,
        mesh=mesh,
        out_type=jax.ShapeDtypeStruct((Q * K_CHUNKS, CHUNK), jnp.float32),
        scratch_types=[
            pltpu.VMEM((CIDROWS, 128), jnp.int32),
            pltpu.VMEM((NG, GR), jnp.int32),
            pltpu.VMEM((GR, CHUNK), jnp.float32),
            pltpu.SemaphoreType.DMA,
        ],
        compiler_params=pltpu.CompilerParams(use_tc_tiling_on_sc=False),
    )(_k3_body)
    return kern(s_flat, cid.reshape(Q * K_CHUNKS // 128, 128))


def _k4_body(gs_ref, cid_ref, os_ref, oi_ref):
    gs = gs_ref[...]
    cid = cid_ref[...]
    gid = (cid[:, :, None] * CHUNK
           + lax.broadcasted_iota(jnp.int32, (BQ4, K_CHUNKS, CHUNK), 2))
    lane16 = lax.broadcasted_iota(jnp.int32, (BQ4, K_CHUNKS), 1)
    os_acc = jnp.full((BQ4, K_CHUNKS), NEG, jnp.float32)
    oi_acc = jnp.zeros((BQ4, K_CHUNKS), jnp.int32)
    for p in range(K_OUT):
        m = jnp.max(jnp.max(gs, axis=2), axis=1)
        sel = jnp.where(gs == m[:, None, None], gid, BIG)
        am = jnp.min(jnp.min(sel, axis=2), axis=1)
        os_acc = jnp.where(lane16 == p, m[:, None], os_acc)
        oi_acc = jnp.where(lane16 == p, am[:, None], oi_acc)
        gs = jnp.where(gid == am[:, None, None], NEG, gs)
    os_ref[...] = os_acc
    oi_ref[...] = oi_acc


def _k4(gs3, cid):
    return pl.pallas_call(
        _k4_body,
        grid=(Q // BQ4,),
        in_specs=[
            pl.BlockSpec((BQ4, K_CHUNKS, CHUNK), lambda qi: (qi, 0, 0)),
            pl.BlockSpec((BQ4, K_CHUNKS), lambda qi: (qi, 0)),
        ],
        out_specs=[
            pl.BlockSpec((BQ4, K_CHUNKS), lambda qi: (qi, 0)),
            pl.BlockSpec((BQ4, K_CHUNKS), lambda qi: (qi, 0)),
        ],
        out_shape=[
            jax.ShapeDtypeStruct((Q, K_CHUNKS), jnp.float32),
            jax.ShapeDtypeStruct((Q, K_CHUNKS), jnp.int32),
        ],
    )(gs3, cid)


def kernel(queries, candidates):
    cand_pad = jnp.pad(candidates, ((0, NPAD - N), (0, 0)))
    s, cm3 = _k1(queries, cand_pad)
    cm2 = cm3.transpose(0, 2, 1).reshape(M, Q)
    cid = _k2(cm2)
    gs = _k3(s.reshape(Q * M, CHUNK), cid)
    ts, ti = _k4(gs.reshape(Q, K_CHUNKS, CHUNK), cid)
    return ts[:, :K_OUT], ti[:, :K_OUT]


# speedup vs baseline: 9.2369x; 2.2296x over previous
import functools

import jax
import jax.numpy as jnp
from jax import lax
from jax.experimental import pallas as pl
from jax.experimental.pallas import tpu as pltpu
from jax.experimental.pallas import tpu_sc as plsc

Q = 4096
D = 128
N = 100000
K_OUT = 10

CHUNK = 128
NPAD = 102400
M = NPAD // CHUNK
K_CHUNKS = 16

BQ1 = 1024
BN1 = 2048
NB1 = NPAD // BN1
CPB = BN1 // CHUNK

BQ2 = 512
BQ4 = 256

NEG = -1.0e30
BIG = 2**30


def _k1_body(q_ref, c_ref, s3_ref, cm_ref):
    ni = pl.program_id(1)
    s = lax.dot_general(
        q_ref[...], c_ref[...],
        dimension_numbers=(((1,), (1,)), ((), ())),
        preferred_element_type=jnp.float32,
    )
    col = ni * BN1 + lax.broadcasted_iota(jnp.int32, (BQ1, BN1), 1)
    s = jnp.where(col < N, s, NEG)
    i0 = lax.broadcasted_iota(jnp.int32, (CPB, BQ1), 0)
    cm_t = jnp.full((CPB, BQ1), NEG, jnp.float32)
    for j in range(CPB):
        sj = s[:, j * CHUNK:(j + 1) * CHUNK]
        s3_ref[j] = sj
        cm_t = jnp.where(i0 == j, jnp.max(sj, axis=1)[None, :], cm_t)
    cm_ref[...] = cm_t


def _k1(queries, cand_pad):
    return pl.pallas_call(
        _k1_body,
        grid=(Q // BQ1, NB1),
        in_specs=[
            pl.BlockSpec((BQ1, D), lambda qi, ni: (qi, 0)),
            pl.BlockSpec((BN1, D), lambda qi, ni: (ni, 0)),
        ],
        out_specs=[
            pl.BlockSpec((CPB, BQ1, CHUNK), lambda qi, ni: (ni, qi, 0)),
            pl.BlockSpec((CPB, BQ1), lambda qi, ni: (ni, qi)),
        ],
        out_shape=[
            jax.ShapeDtypeStruct((M, Q, CHUNK), jnp.float32),
            jax.ShapeDtypeStruct((M, Q), jnp.float32),
        ],
    )(queries, cand_pad)


def _k2_body(cm_ref, cid_ref):
    cm = cm_ref[...]
    gid0 = lax.broadcasted_iota(jnp.int32, (M, BQ2), 0)
    lane16 = lax.broadcasted_iota(jnp.int32, (BQ2, K_CHUNKS), 1)
    cid_acc = jnp.zeros((BQ2, K_CHUNKS), jnp.int32)
    for p in range(K_CHUNKS):
        m = jnp.max(cm, axis=0)
        sel = jnp.where(cm == m[None, :], gid0, BIG)
        am = jnp.min(sel, axis=0)
        cid_acc = jnp.where(lane16 == p, am[:, None], cid_acc)
        cm = jnp.where(gid0 == am[None, :], NEG, cm)
    cid_ref[...] = cid_acc


def _k2(cm2):
    return pl.pallas_call(
        _k2_body,
        grid=(Q // BQ2,),
        in_specs=[pl.BlockSpec((M, BQ2), lambda qi: (0, qi))],
        out_specs=pl.BlockSpec((BQ2, K_CHUNKS), lambda qi: (qi, 0)),
        out_shape=jax.ShapeDtypeStruct((Q, K_CHUNKS), jnp.int32),
    )(cm2)


NW = 32
QPW = Q // NW
RPW = QPW * K_CHUNKS
NG = 8
GR = RPW // NG
MPG = QPW // NG


CIDROWS = QPW * K_CHUNKS // 128


def _k3_body(s_hbm, cid_hbm, out_hbm, cid_v, idx_v, rows_v, sem):
    wid = lax.axis_index("s") * 2 + lax.axis_index("c")
    qlo = wid * QPW
    pltpu.sync_copy(cid_hbm.at[pl.ds(wid * CIDROWS, CIDROWS)], cid_v)

    def fill(i, carry):
        flat = cid_v[i // 8, pl.ds((i % 8) * K_CHUNKS, K_CHUNKS)]
        flat = flat * Q + (qlo + i)
        g = i // MPG
        off = (i % MPG) * K_CHUNKS
        idx_v[g, pl.ds(off, K_CHUNKS)] = flat
        return carry

    lax.fori_loop(0, QPW, fill, 0)
    for g in range(NG):
        pltpu.async_copy(s_hbm.at[idx_v.at[g]], rows_v, sem).wait()
        pltpu.sync_copy(rows_v,
                        out_hbm.at[pl.ds(wid * RPW + g * GR, GR)])


def _k3(s_flat, cid):
    mesh = plsc.VectorSubcoreMesh(core_axis_name="c", subcore_axis_name="s")
    kern = functools.partial(
        pl.kernel,
        mesh=mesh,
        out_type=jax.ShapeDtypeStruct((Q * K_CHUNKS, CHUNK), jnp.float32),
        scratch_types=[
            pltpu.VMEM((CIDROWS, 128), jnp.int32),
            pltpu.VMEM((NG, GR), jnp.int32),
            pltpu.VMEM((GR, CHUNK), jnp.float32),
            pltpu.SemaphoreType.DMA,
        ],
        compiler_params=pltpu.CompilerParams(use_tc_tiling_on_sc=False),
    )(_k3_body)
    return kern(s_flat, cid.reshape(Q * K_CHUNKS // 128, 128))


def _k4_body(gs_ref, cid_ref, os_ref, oi_ref):
    gs = gs_ref[...]
    cid = cid_ref[...]
    gid = (cid[:, :, None] * CHUNK
           + lax.broadcasted_iota(jnp.int32, (BQ4, K_CHUNKS, CHUNK), 2))
    lane16 = lax.broadcasted_iota(jnp.int32, (BQ4, K_CHUNKS), 1)
    os_acc = jnp.full((BQ4, K_CHUNKS), NEG, jnp.float32)
    oi_acc = jnp.zeros((BQ4, K_CHUNKS), jnp.int32)
    for p in range(K_OUT):
        m = jnp.max(jnp.max(gs, axis=2), axis=1)
        sel = jnp.where(gs == m[:, None, None], gid, BIG)
        am = jnp.min(jnp.min(sel, axis=2), axis=1)
        os_acc = jnp.where(lane16 == p, m[:, None], os_acc)
        oi_acc = jnp.where(lane16 == p, am[:, None], oi_acc)
        gs = jnp.where(gid == am[:, None, None], NEG, gs)
    os_ref[...] = os_acc
    oi_ref[...] = oi_acc


def _k4(gs3, cid):
    return pl.pallas_call(
        _k4_body,
        grid=(Q // BQ4,),
        in_specs=[
            pl.BlockSpec((BQ4, K_CHUNKS, CHUNK), lambda qi: (qi, 0, 0)),
            pl.BlockSpec((BQ4, K_CHUNKS), lambda qi: (qi, 0)),
        ],
        out_specs=[
            pl.BlockSpec((BQ4, K_CHUNKS), lambda qi: (qi, 0)),
            pl.BlockSpec((BQ4, K_CHUNKS), lambda qi: (qi, 0)),
        ],
        out_shape=[
            jax.ShapeDtypeStruct((Q, K_CHUNKS), jnp.float32),
            jax.ShapeDtypeStruct((Q, K_CHUNKS), jnp.int32),
        ],
    )(gs3, cid)


def kernel(queries, candidates):
    cand_pad = jnp.pad(candidates, ((0, NPAD - N), (0, 0)))
    s3, cm2 = _k1(queries, cand_pad)
    cid = _k2(cm2)
    gs = _k3(s3.reshape(M * Q, CHUNK), cid)
    ts, ti = _k4(gs.reshape(Q, K_CHUNKS, CHUNK), cid)
    return ts[:, :K_OUT], ti[:, :K_OUT]


# speedup vs baseline: 10.8615x; 1.1759x over previous
import functools

import jax
import jax.numpy as jnp
from jax import lax
from jax.experimental import pallas as pl
from jax.experimental.pallas import tpu as pltpu
from jax.experimental.pallas import tpu_sc as plsc

Q = 4096
D = 128
N = 100000
K_OUT = 10

CHUNK = 128
NPAD = 102400
M = NPAD // CHUNK
K_CHUNKS = 16

BQ1 = 1024
BN1 = 2048
NB1 = NPAD // BN1
CPB = BN1 // CHUNK

BQ2 = 512
BQ4 = 512

NEG = -1.0e30
BIG = 2**30


def _k1_body(q_ref, c_ref, s3_ref, cm_ref):
    ni = pl.program_id(1)
    s = lax.dot_general(
        q_ref[...], c_ref[...],
        dimension_numbers=(((1,), (1,)), ((), ())),
        preferred_element_type=jnp.float32,
    )
    col = ni * BN1 + lax.broadcasted_iota(jnp.int32, (BQ1, BN1), 1)
    s = jnp.where(col < N, s, NEG)
    i0 = lax.broadcasted_iota(jnp.int32, (CPB, BQ1), 0)
    cm_t = jnp.full((CPB, BQ1), NEG, jnp.float32)
    for j in range(CPB):
        sj = s[:, j * CHUNK:(j + 1) * CHUNK]
        s3_ref[j] = sj
        cm_t = jnp.where(i0 == j, jnp.max(sj, axis=1)[None, :], cm_t)
    cm_ref[...] = cm_t


def _k1(queries, cand_pad):
    return pl.pallas_call(
        _k1_body,
        grid=(Q // BQ1, NB1),
        in_specs=[
            pl.BlockSpec((BQ1, D), lambda qi, ni: (qi, 0)),
            pl.BlockSpec((BN1, D), lambda qi, ni: (ni, 0)),
        ],
        out_specs=[
            pl.BlockSpec((CPB, BQ1, CHUNK), lambda qi, ni: (ni, qi, 0)),
            pl.BlockSpec((CPB, BQ1), lambda qi, ni: (ni, qi)),
        ],
        out_shape=[
            jax.ShapeDtypeStruct((M, Q, CHUNK), jnp.float32),
            jax.ShapeDtypeStruct((M, Q), jnp.float32),
        ],
    )(queries, cand_pad)


def _k2_body(cm_ref, cid_ref):
    cm = cm_ref[...]
    gid0 = lax.broadcasted_iota(jnp.int32, (M, BQ2), 0)
    lane16 = lax.broadcasted_iota(jnp.int32, (BQ2, K_CHUNKS), 1)
    cid_acc = jnp.zeros((BQ2, K_CHUNKS), jnp.int32)
    for p in range(K_CHUNKS):
        m = jnp.max(cm, axis=0)
        sel = jnp.where(cm == m[None, :], gid0, BIG)
        am = jnp.min(sel, axis=0)
        cid_acc = jnp.where(lane16 == p, am[:, None], cid_acc)
        cm = jnp.where(gid0 == am[None, :], NEG, cm)
    cid_ref[...] = cid_acc


def _k2(cm2):
    return pl.pallas_call(
        _k2_body,
        grid=(Q // BQ2,),
        in_specs=[pl.BlockSpec((M, BQ2), lambda qi: (0, qi))],
        out_specs=pl.BlockSpec((BQ2, K_CHUNKS), lambda qi: (qi, 0)),
        out_shape=jax.ShapeDtypeStruct((Q, K_CHUNKS), jnp.int32),
    )(cm2)


NW = 32
QPW = Q // NW
RPW = QPW * K_CHUNKS
NG = 8
GR = RPW // NG
MPG = QPW // NG


CIDROWS = QPW * K_CHUNKS // 128


def _k3_body(s_hbm, cid_hbm, out_hbm, cid_v, idx_v, rows_v, sem):
    wid = lax.axis_index("s") * 2 + lax.axis_index("c")
    qlo = wid * QPW
    pltpu.sync_copy(cid_hbm.at[pl.ds(wid * CIDROWS, CIDROWS)], cid_v)

    def fill(i, carry):
        flat = cid_v[i // 8, pl.ds((i % 8) * K_CHUNKS, K_CHUNKS)]
        flat = flat * Q + (qlo + i)
        g = i // MPG
        off = (i % MPG) * K_CHUNKS
        idx_v[g, pl.ds(off, K_CHUNKS)] = flat
        return carry

    lax.fori_loop(0, QPW, fill, 0)
    for g in range(NG):
        pltpu.async_copy(s_hbm.at[idx_v.at[g]], rows_v, sem).wait()
        pltpu.sync_copy(rows_v,
                        out_hbm.at[pl.ds(wid * RPW + g * GR, GR)])


def _k3(s_flat, cid):
    mesh = plsc.VectorSubcoreMesh(core_axis_name="c", subcore_axis_name="s")
    kern = functools.partial(
        pl.kernel,
        mesh=mesh,
        out_type=jax.ShapeDtypeStruct((Q * K_CHUNKS, CHUNK), jnp.float32),
        scratch_types=[
            pltpu.VMEM((CIDROWS, 128), jnp.int32),
            pltpu.VMEM((NG, GR), jnp.int32),
            pltpu.VMEM((GR, CHUNK), jnp.float32),
            pltpu.SemaphoreType.DMA,
        ],
        compiler_params=pltpu.CompilerParams(use_tc_tiling_on_sc=False),
    )(_k3_body)
    return kern(s_flat, cid.reshape(Q * K_CHUNKS // 128, 128))


GROWS = K_CHUNKS * CHUNK


def _k4_body(gt_ref, cidt_ref, os_ref, oi_ref):
    gt = gt_ref[...]
    cidt = cidt_ref[...]
    cid_rows = jnp.broadcast_to(
        cidt.reshape(K_CHUNKS, 1, BQ4), (K_CHUNKS, CHUNK, BQ4)
    ).reshape(GROWS, BQ4)
    lane = jnp.bitwise_and(
        lax.broadcasted_iota(jnp.int32, (GROWS, BQ4), 0), CHUNK - 1)
    gid = cid_rows * CHUNK + lane
    i16 = lax.broadcasted_iota(jnp.int32, (K_CHUNKS, BQ4), 0)
    os_acc = jnp.full((K_CHUNKS, BQ4), NEG, jnp.float32)
    oi_acc = jnp.zeros((K_CHUNKS, BQ4), jnp.int32)
    for p in range(K_OUT):
        m = jnp.max(gt, axis=0)
        sel = jnp.where(gt == m[None, :], gid, BIG)
        am = jnp.min(sel, axis=0)
        os_acc = jnp.where(i16 == p, m[None, :], os_acc)
        oi_acc = jnp.where(i16 == p, am[None, :], oi_acc)
        gt = jnp.where(gid == am[None, :], NEG, gt)
    os_ref[...] = os_acc
    oi_ref[...] = oi_acc


def _k4(gt, cidt):
    return pl.pallas_call(
        _k4_body,
        grid=(Q // BQ4,),
        in_specs=[
            pl.BlockSpec((GROWS, BQ4), lambda qi: (0, qi)),
            pl.BlockSpec((K_CHUNKS, BQ4), lambda qi: (0, qi)),
        ],
        out_specs=[
            pl.BlockSpec((K_CHUNKS, BQ4), lambda qi: (0, qi)),
            pl.BlockSpec((K_CHUNKS, BQ4), lambda qi: (0, qi)),
        ],
        out_shape=[
            jax.ShapeDtypeStruct((K_CHUNKS, Q), jnp.float32),
            jax.ShapeDtypeStruct((K_CHUNKS, Q), jnp.int32),
        ],
    )(gt, cidt)


def kernel(queries, candidates):
    cand_pad = jnp.pad(candidates, ((0, NPAD - N), (0, 0)))
    s3, cm2 = _k1(queries, cand_pad)
    cid = _k2(cm2)
    gs = _k3(s3.reshape(M * Q, CHUNK), cid)
    gt = gs.reshape(Q, GROWS).T
    ts, ti = _k4(gt, cid.T)
    return ts.T[:, :K_OUT], ti.T[:, :K_OUT]


# speedup vs baseline: 12.2611x; 1.1289x over previous
import functools

import jax
import jax.numpy as jnp
from jax import lax
from jax.experimental import pallas as pl
from jax.experimental.pallas import tpu as pltpu
from jax.experimental.pallas import tpu_sc as plsc

Q = 4096
D = 128
N = 100000
K_OUT = 10

CHUNK = 128
NPAD = 100352
M = NPAD // CHUNK
K_CHUNKS = 16

BQ1 = 4096
BN1 = 1024
NB1 = NPAD // BN1
CPB = BN1 // CHUNK

BQ2 = 512
BQ4 = 512

NEG = -1.0e30
BIG = 2**30


def _k1_body(q_ref, c_ref, s3_ref, cm_ref):
    ni = pl.program_id(1)
    s = lax.dot_general(
        q_ref[...], c_ref[...],
        dimension_numbers=(((1,), (1,)), ((), ())),
        preferred_element_type=jnp.float32,
    )
    col = ni * BN1 + lax.broadcasted_iota(jnp.int32, (BQ1, BN1), 1)
    s = jnp.where(col < N, s, NEG)
    i0 = lax.broadcasted_iota(jnp.int32, (CPB, BQ1), 0)
    cm_t = jnp.full((CPB, BQ1), NEG, jnp.float32)
    for j in range(CPB):
        sj = s[:, j * CHUNK:(j + 1) * CHUNK]
        s3_ref[j] = sj
        cm_t = jnp.where(i0 == j, jnp.max(sj, axis=1)[None, :], cm_t)
    cm_ref[...] = cm_t


def _k1(queries, cand_pad):
    return pl.pallas_call(
        _k1_body,
        grid=(Q // BQ1, NB1),
        in_specs=[
            pl.BlockSpec((BQ1, D), lambda qi, ni: (qi, 0)),
            pl.BlockSpec((BN1, D), lambda qi, ni: (ni, 0)),
        ],
        out_specs=[
            pl.BlockSpec((CPB, BQ1, CHUNK), lambda qi, ni: (ni, qi, 0)),
            pl.BlockSpec((CPB, BQ1), lambda qi, ni: (ni, qi)),
        ],
        out_shape=[
            jax.ShapeDtypeStruct((M, Q, CHUNK), jnp.float32),
            jax.ShapeDtypeStruct((M, Q), jnp.float32),
        ],
    )(queries, cand_pad)


def _k2_body(cm_ref, cid_ref):
    cm = cm_ref[...]
    gid0 = lax.broadcasted_iota(jnp.int32, (M, BQ2), 0)
    lane16 = lax.broadcasted_iota(jnp.int32, (BQ2, K_CHUNKS), 1)
    cid_acc = jnp.zeros((BQ2, K_CHUNKS), jnp.int32)
    for p in range(K_CHUNKS):
        m = jnp.max(cm, axis=0)
        sel = jnp.where(cm == m[None, :], gid0, BIG)
        am = jnp.min(sel, axis=0)
        cid_acc = jnp.where(lane16 == p, am[:, None], cid_acc)
        cm = jnp.where(gid0 == am[None, :], NEG, cm)
    cid_ref[...] = cid_acc


def _k2(cm2):
    return pl.pallas_call(
        _k2_body,
        grid=(Q // BQ2,),
        in_specs=[pl.BlockSpec((M, BQ2), lambda qi: (0, qi))],
        out_specs=pl.BlockSpec((BQ2, K_CHUNKS), lambda qi: (qi, 0)),
        out_shape=jax.ShapeDtypeStruct((Q, K_CHUNKS), jnp.int32),
    )(cm2)


NW = 32
QPW = Q // NW
RPW = QPW * K_CHUNKS
NG = 8
GR = RPW // NG
MPG = QPW // NG


CIDROWS = QPW * K_CHUNKS // 128


def _k3_body(s_hbm, cid_hbm, out_hbm, cid_v, idx_v, rows_v, sem):
    wid = lax.axis_index("s") * 2 + lax.axis_index("c")
    qlo = wid * QPW
    pltpu.sync_copy(cid_hbm.at[pl.ds(wid * CIDROWS, CIDROWS)], cid_v)

    def fill(i, carry):
        flat = cid_v[i // 8, pl.ds((i % 8) * K_CHUNKS, K_CHUNKS)]
        flat = flat * Q + (qlo + i)
        g = i // MPG
        off = (i % MPG) * K_CHUNKS
        idx_v[g, pl.ds(off, K_CHUNKS)] = flat
        return carry

    lax.fori_loop(0, QPW, fill, 0)
    for g in range(NG):
        pltpu.async_copy(s_hbm.at[idx_v.at[g]], rows_v, sem).wait()
        pltpu.sync_copy(rows_v,
                        out_hbm.at[pl.ds(wid * RPW + g * GR, GR)])


def _k3(s_flat, cid):
    mesh = plsc.VectorSubcoreMesh(core_axis_name="c", subcore_axis_name="s")
    kern = functools.partial(
        pl.kernel,
        mesh=mesh,
        out_type=jax.ShapeDtypeStruct((Q * K_CHUNKS, CHUNK), jnp.float32),
        scratch_types=[
            pltpu.VMEM((CIDROWS, 128), jnp.int32),
            pltpu.VMEM((NG, GR), jnp.int32),
            pltpu.VMEM((GR, CHUNK), jnp.float32),
            pltpu.SemaphoreType.DMA,
        ],
        compiler_params=pltpu.CompilerParams(use_tc_tiling_on_sc=False),
    )(_k3_body)
    return kern(s_flat, cid.reshape(Q * K_CHUNKS // 128, 128))


GROWS = K_CHUNKS * CHUNK


def _k4_body(gt_ref, cidt_ref, os_ref, oi_ref):
    gt = gt_ref[...]
    cidt = cidt_ref[...]
    cid_rows = jnp.broadcast_to(
        cidt.reshape(K_CHUNKS, 1, BQ4), (K_CHUNKS, CHUNK, BQ4)
    ).reshape(GROWS, BQ4)
    lane = jnp.bitwise_and(
        lax.broadcasted_iota(jnp.int32, (GROWS, BQ4), 0), CHUNK - 1)
    gid = cid_rows * CHUNK + lane
    i16 = lax.broadcasted_iota(jnp.int32, (K_CHUNKS, BQ4), 0)
    os_acc = jnp.full((K_CHUNKS, BQ4), NEG, jnp.float32)
    oi_acc = jnp.zeros((K_CHUNKS, BQ4), jnp.int32)
    for p in range(K_OUT):
        m = jnp.max(gt, axis=0)
        sel = jnp.where(gt == m[None, :], gid, BIG)
        am = jnp.min(sel, axis=0)
        os_acc = jnp.where(i16 == p, m[None, :], os_acc)
        oi_acc = jnp.where(i16 == p, am[None, :], oi_acc)
        gt = jnp.where(gid == am[None, :], NEG, gt)
    os_ref[...] = os_acc
    oi_ref[...] = oi_acc


def _k4(gt, cidt):
    return pl.pallas_call(
        _k4_body,
        grid=(Q // BQ4,),
        in_specs=[
            pl.BlockSpec((GROWS, BQ4), lambda qi: (0, qi)),
            pl.BlockSpec((K_CHUNKS, BQ4), lambda qi: (0, qi)),
        ],
        out_specs=[
            pl.BlockSpec((K_CHUNKS, BQ4), lambda qi: (0, qi)),
            pl.BlockSpec((K_CHUNKS, BQ4), lambda qi: (0, qi)),
        ],
        out_shape=[
            jax.ShapeDtypeStruct((K_CHUNKS, Q), jnp.float32),
            jax.ShapeDtypeStruct((K_CHUNKS, Q), jnp.int32),
        ],
    )(gt, cidt)


def kernel(queries, candidates):
    cand_pad = jnp.pad(candidates, ((0, NPAD - N), (0, 0)))
    s3, cm2 = _k1(queries, cand_pad)
    cid = _k2(cm2)
    gs = _k3(s3.reshape(M * Q, CHUNK), cid)
    gt = gs.reshape(Q, GROWS).T
    ts, ti = _k4(gt, cid.T)
    return ts.T[:, :K_OUT], ti.T[:, :K_OUT]


# speedup vs baseline: 12.9533x; 1.0565x over previous
import functools

import jax
import jax.numpy as jnp
from jax import lax
from jax.experimental import pallas as pl
from jax.experimental.pallas import tpu as pltpu
from jax.experimental.pallas import tpu_sc as plsc

Q = 4096
D = 128
N = 100000
K_OUT = 10

CHUNK = 128
NPAD = 100352
M = NPAD // CHUNK
K_CHUNKS = 16

BQ1 = 4096
BN1 = 1024
NB1 = NPAD // BN1
CPB = BN1 // CHUNK

BQ2 = 512
BQ4 = 512

NEG = -1.0e30
BIG = 2**30


def _k1_body(q_ref, c_ref, s3_ref, cm_ref):
    ni = pl.program_id(1)
    s = lax.dot_general(
        q_ref[...], c_ref[...],
        dimension_numbers=(((1,), (1,)), ((), ())),
        preferred_element_type=jnp.float32,
    )
    col = ni * BN1 + lax.broadcasted_iota(jnp.int32, (BQ1, BN1), 1)
    s = jnp.where(col < N, s, NEG)
    i0 = lax.broadcasted_iota(jnp.int32, (CPB, BQ1), 0)
    cm_t = jnp.full((CPB, BQ1), NEG, jnp.float32)
    for j in range(CPB):
        sj = s[:, j * CHUNK:(j + 1) * CHUNK]
        s3_ref[j] = sj
        cm_t = jnp.where(i0 == j, jnp.max(sj, axis=1)[None, :], cm_t)
    cm_ref[...] = cm_t


def _k1(queries, cand_pad):
    return pl.pallas_call(
        _k1_body,
        grid=(Q // BQ1, NB1),
        in_specs=[
            pl.BlockSpec((BQ1, D), lambda qi, ni: (qi, 0)),
            pl.BlockSpec((BN1, D), lambda qi, ni: (ni, 0)),
        ],
        out_specs=[
            pl.BlockSpec((CPB, BQ1, CHUNK), lambda qi, ni: (ni, qi, 0)),
            pl.BlockSpec((CPB, BQ1), lambda qi, ni: (ni, qi)),
        ],
        out_shape=[
            jax.ShapeDtypeStruct((M, Q, CHUNK), jnp.float32),
            jax.ShapeDtypeStruct((M, Q), jnp.float32),
        ],
    )(queries, cand_pad)


def _k2_body(cm_ref, cid_ref):
    cm = cm_ref[...]
    gid0 = lax.broadcasted_iota(jnp.int32, (M, BQ2), 0)
    lane16 = lax.broadcasted_iota(jnp.int32, (BQ2, K_CHUNKS), 1)
    cid_acc = jnp.zeros((BQ2, K_CHUNKS), jnp.int32)
    for p in range(K_CHUNKS):
        m = jnp.max(cm, axis=0)
        sel = jnp.where(cm == m[None, :], gid0, BIG)
        am = jnp.min(sel, axis=0)
        cid_acc = jnp.where(lane16 == p, am[:, None], cid_acc)
        cm = jnp.where(gid0 == am[None, :], NEG, cm)
    cid_ref[...] = cid_acc


def _k2(cm2):
    return pl.pallas_call(
        _k2_body,
        grid=(Q // BQ2,),
        in_specs=[pl.BlockSpec((M, BQ2), lambda qi: (0, qi))],
        out_specs=pl.BlockSpec((BQ2, K_CHUNKS), lambda qi: (qi, 0)),
        out_shape=jax.ShapeDtypeStruct((Q, K_CHUNKS), jnp.int32),
    )(cm2)


NW = 32
QPW = Q // NW
RPW = QPW * K_CHUNKS
NG = 8
GR = RPW // NG
MPG = QPW // NG


CIDROWS = QPW * K_CHUNKS // 128


def _k3_body(s_hbm, cid_hbm, out_hbm, cid_v, idx_v, rows_v, sem):
    wid = lax.axis_index("s") * 2 + lax.axis_index("c")
    qlo = wid * QPW
    pltpu.sync_copy(cid_hbm.at[pl.ds(wid * CIDROWS, CIDROWS)], cid_v)

    def fill(i, carry):
        flat = cid_v[i // 8, pl.ds((i % 8) * K_CHUNKS, K_CHUNKS)]
        flat = flat * Q + (qlo + i)
        g = i // MPG
        off = (i % MPG) * K_CHUNKS
        idx_v[g, pl.ds(off, K_CHUNKS)] = flat
        return carry

    lax.fori_loop(0, QPW, fill, 0)
    for g in range(NG):
        pltpu.async_copy(s_hbm.at[idx_v.at[g]], rows_v, sem).wait()
        pltpu.sync_copy(rows_v,
                        out_hbm.at[pl.ds(wid * RPW + g * GR, GR)])


def _k3(s_flat, cid):
    mesh = plsc.VectorSubcoreMesh(core_axis_name="c", subcore_axis_name="s")
    kern = functools.partial(
        pl.kernel,
        mesh=mesh,
        out_type=jax.ShapeDtypeStruct((Q * K_CHUNKS, CHUNK), jnp.float32),
        scratch_types=[
            pltpu.VMEM((CIDROWS, 128), jnp.int32),
            pltpu.VMEM((NG, GR), jnp.int32),
            pltpu.VMEM((GR, CHUNK), jnp.float32),
            pltpu.SemaphoreType.DMA,
        ],
        compiler_params=pltpu.CompilerParams(use_tc_tiling_on_sc=False),
    )(_k3_body)
    return kern(s_flat, cid.reshape(Q * K_CHUNKS // 128, 128))


GROWS = K_CHUNKS * CHUNK


def _k4_body(gs_ref, cidt_ref, os_ref, oi_ref):
    gt = jnp.concatenate(
        [gs_ref[:, j, :].T for j in range(K_CHUNKS)], axis=0)
    cidt = cidt_ref[...]
    cid_rows = jnp.broadcast_to(
        cidt.reshape(K_CHUNKS, 1, BQ4), (K_CHUNKS, CHUNK, BQ4)
    ).reshape(GROWS, BQ4)
    lane = jnp.bitwise_and(
        lax.broadcasted_iota(jnp.int32, (GROWS, BQ4), 0), CHUNK - 1)
    gid = cid_rows * CHUNK + lane
    i16 = lax.broadcasted_iota(jnp.int32, (K_CHUNKS, BQ4), 0)
    os_acc = jnp.full((K_CHUNKS, BQ4), NEG, jnp.float32)
    oi_acc = jnp.zeros((K_CHUNKS, BQ4), jnp.int32)
    for p in range(K_OUT):
        m = jnp.max(gt, axis=0)
        sel = jnp.where(gt == m[None, :], gid, BIG)
        am = jnp.min(sel, axis=0)
        os_acc = jnp.where(i16 == p, m[None, :], os_acc)
        oi_acc = jnp.where(i16 == p, am[None, :], oi_acc)
        gt = jnp.where(gid == am[None, :], NEG, gt)
    os_ref[...] = os_acc
    oi_ref[...] = oi_acc


def _k4(gs3, cidt):
    return pl.pallas_call(
        _k4_body,
        grid=(Q // BQ4,),
        in_specs=[
            pl.BlockSpec((BQ4, K_CHUNKS, CHUNK), lambda qi: (qi, 0, 0)),
            pl.BlockSpec((K_CHUNKS, BQ4), lambda qi: (0, qi)),
        ],
        out_specs=[
            pl.BlockSpec((K_CHUNKS, BQ4), lambda qi: (0, qi)),
            pl.BlockSpec((K_CHUNKS, BQ4), lambda qi: (0, qi)),
        ],
        out_shape=[
            jax.ShapeDtypeStruct((K_CHUNKS, Q), jnp.float32),
            jax.ShapeDtypeStruct((K_CHUNKS, Q), jnp.int32),
        ],
    )(gs3, cidt)


def kernel(queries, candidates):
    cand_pad = jnp.pad(candidates, ((0, NPAD - N), (0, 0)))
    s3, cm2 = _k1(queries, cand_pad)
    cid = _k2(cm2)
    gs = _k3(s3.reshape(M * Q, CHUNK), cid)
    ts, ti = _k4(gs.reshape(Q, K_CHUNKS, CHUNK), cid.T)
    return ts.T[:, :K_OUT], ti.T[:, :K_OUT]


# speedup vs baseline: 13.0294x; 1.0059x over previous
import functools

import jax
import jax.numpy as jnp
from jax import lax
from jax.experimental import pallas as pl
from jax.experimental.pallas import tpu as pltpu
from jax.experimental.pallas import tpu_sc as plsc

Q = 4096
D = 128
N = 100000
K_OUT = 10

CHUNK = 128
NPAD = 100352
M = NPAD // CHUNK
K_CHUNKS = 16

BQ1 = 4096
BN1 = 1024
NB1 = NPAD // BN1
CPB = BN1 // CHUNK

BQ2 = 512
BQ4 = 512

NEG = -1.0e30
BIG = 2**30


def _k1_body(q_ref, c_ref, s3_ref, cm_ref):
    ni = pl.program_id(1)
    s = lax.dot_general(
        q_ref[...], c_ref[...],
        dimension_numbers=(((1,), (1,)), ((), ())),
        preferred_element_type=jnp.float32,
    )
    col = ni * BN1 + lax.broadcasted_iota(jnp.int32, (BQ1, BN1), 1)
    s = jnp.where(col < N, s, NEG)
    i0 = lax.broadcasted_iota(jnp.int32, (CPB, BQ1), 0)
    cm_t = jnp.full((CPB, BQ1), NEG, jnp.float32)
    for j in range(CPB):
        sj = s[:, j * CHUNK:(j + 1) * CHUNK]
        s3_ref[j] = sj
        cm_t = jnp.where(i0 == j, jnp.max(sj, axis=1)[None, :], cm_t)
    cm_ref[...] = cm_t


def _k1(queries, cand_pad):
    return pl.pallas_call(
        _k1_body,
        grid=(Q // BQ1, NB1),
        in_specs=[
            pl.BlockSpec((BQ1, D), lambda qi, ni: (qi, 0)),
            pl.BlockSpec((BN1, D), lambda qi, ni: (ni, 0)),
        ],
        out_specs=[
            pl.BlockSpec((CPB, BQ1, CHUNK), lambda qi, ni: (ni, qi, 0)),
            pl.BlockSpec((CPB, BQ1), lambda qi, ni: (ni, qi)),
        ],
        out_shape=[
            jax.ShapeDtypeStruct((M, Q, CHUNK), jnp.float32),
            jax.ShapeDtypeStruct((M, Q), jnp.float32),
        ],
    )(queries, cand_pad)


def _k2_body(cm_ref, cid_ref):
    cm = cm_ref[...]
    gid0 = lax.broadcasted_iota(jnp.int32, (M, BQ2), 0)
    lane16 = lax.broadcasted_iota(jnp.int32, (BQ2, K_CHUNKS), 1)
    cid_acc = jnp.zeros((BQ2, K_CHUNKS), jnp.int32)
    for p in range(K_CHUNKS):
        m = jnp.max(cm, axis=0)
        sel = jnp.where(cm == m[None, :], gid0, BIG)
        am = jnp.min(sel, axis=0)
        cid_acc = jnp.where(lane16 == p, am[:, None], cid_acc)
        cm = jnp.where(gid0 == am[None, :], NEG, cm)
    cid_ref[...] = cid_acc


HQ = Q // 2


def _k2(cm2, qh):
    nb = HQ // BQ2
    return pl.pallas_call(
        _k2_body,
        grid=(nb,),
        in_specs=[pl.BlockSpec((M, BQ2), lambda qi: (0, qi + qh * nb))],
        out_specs=pl.BlockSpec((BQ2, K_CHUNKS), lambda qi: (qi, 0)),
        out_shape=jax.ShapeDtypeStruct((HQ, K_CHUNKS), jnp.int32),
    )(cm2)


NW = 32
QPW = HQ // NW
RPW = QPW * K_CHUNKS
NG = 4
GR = RPW // NG
MPG = QPW // NG
CIDROWS = RPW // 128


def _make_k3_body(qbase):
    def body(s_hbm, cid_hbm, out_hbm, cid_v, idx_v, rows_v, sem):
        wid = lax.axis_index("s") * 2 + lax.axis_index("c")
        qlo = wid * QPW
        pltpu.sync_copy(cid_hbm.at[pl.ds(wid * CIDROWS, CIDROWS)], cid_v)

        def fill(i, carry):
            flat = cid_v[i // 8, pl.ds((i % 8) * K_CHUNKS, K_CHUNKS)]
            flat = flat * Q + (qbase + qlo + i)
            g = i // MPG
            off = (i % MPG) * K_CHUNKS
            idx_v[g, pl.ds(off, K_CHUNKS)] = flat
            return carry

        lax.fori_loop(0, QPW, fill, 0)
        for g in range(NG):
            pltpu.async_copy(s_hbm.at[idx_v.at[g]], rows_v, sem).wait()
            pltpu.sync_copy(rows_v,
                            out_hbm.at[pl.ds(wid * RPW + g * GR, GR)])
    return body


def _k3(s_flat, cid_h, qbase):
    mesh = plsc.VectorSubcoreMesh(core_axis_name="c", subcore_axis_name="s")
    kern = functools.partial(
        pl.kernel,
        mesh=mesh,
        out_type=jax.ShapeDtypeStruct((HQ * K_CHUNKS, CHUNK), jnp.float32),
        scratch_types=[
            pltpu.VMEM((CIDROWS, 128), jnp.int32),
            pltpu.VMEM((NG, GR), jnp.int32),
            pltpu.VMEM((GR, CHUNK), jnp.float32),
            pltpu.SemaphoreType.DMA,
        ],
        compiler_params=pltpu.CompilerParams(use_tc_tiling_on_sc=False),
    )(_make_k3_body(qbase))
    return kern(s_flat, cid_h.reshape(HQ * K_CHUNKS // 128, 128))


GROWS = K_CHUNKS * CHUNK


def _k4_body(gs_ref, cidt_ref, os_ref, oi_ref):
    gt = jnp.concatenate(
        [gs_ref[:, j, :].T for j in range(K_CHUNKS)], axis=0)
    cidt = cidt_ref[...]
    cid_rows = jnp.broadcast_to(
        cidt.reshape(K_CHUNKS, 1, BQ4), (K_CHUNKS, CHUNK, BQ4)
    ).reshape(GROWS, BQ4)
    lane = jnp.bitwise_and(
        lax.broadcasted_iota(jnp.int32, (GROWS, BQ4), 0), CHUNK - 1)
    gid = cid_rows * CHUNK + lane
    i16 = lax.broadcasted_iota(jnp.int32, (K_CHUNKS, BQ4), 0)
    os_acc = jnp.full((K_CHUNKS, BQ4), NEG, jnp.float32)
    oi_acc = jnp.zeros((K_CHUNKS, BQ4), jnp.int32)
    for p in range(K_OUT):
        m = jnp.max(gt, axis=0)
        sel = jnp.where(gt == m[None, :], gid, BIG)
        am = jnp.min(sel, axis=0)
        os_acc = jnp.where(i16 == p, m[None, :], os_acc)
        oi_acc = jnp.where(i16 == p, am[None, :], oi_acc)
        gt = jnp.where(gid == am[None, :], NEG, gt)
    os_ref[...] = os_acc
    oi_ref[...] = oi_acc


def _k4(gs3_h, cidt_h):
    return pl.pallas_call(
        _k4_body,
        grid=(HQ // BQ4,),
        in_specs=[
            pl.BlockSpec((BQ4, K_CHUNKS, CHUNK), lambda qi: (qi, 0, 0)),
            pl.BlockSpec((K_CHUNKS, BQ4), lambda qi: (0, qi)),
        ],
        out_specs=[
            pl.BlockSpec((K_CHUNKS, BQ4), lambda qi: (0, qi)),
            pl.BlockSpec((K_CHUNKS, BQ4), lambda qi: (0, qi)),
        ],
        out_shape=[
            jax.ShapeDtypeStruct((K_CHUNKS, HQ), jnp.float32),
            jax.ShapeDtypeStruct((K_CHUNKS, HQ), jnp.int32),
        ],
    )(gs3_h, cidt_h)


def kernel(queries, candidates):
    cand_pad = jnp.pad(candidates, ((0, NPAD - N), (0, 0)))
    s3, cm2 = _k1(queries, cand_pad)
    s_flat = s3.reshape(M * Q, CHUNK)
    ts_h, ti_h = [], []
    for h in range(2):
        cid_h = _k2(cm2, h)
        gs_h = _k3(s_flat, cid_h, h * HQ)
        t_s, t_i = _k4(gs_h.reshape(HQ, K_CHUNKS, CHUNK), cid_h.T)
        ts_h.append(t_s)
        ti_h.append(t_i)
    ts = jnp.concatenate(ts_h, axis=1)
    ti = jnp.concatenate(ti_h, axis=1)
    return ts.T[:, :K_OUT], ti.T[:, :K_OUT]
